# baseline (speedup 1.0000x reference)
import functools
import math

import jax
import jax.numpy as jnp
from jax import lax
from jax.experimental import pallas as pl
from jax.experimental.pallas import tpu as pltpu
from jax.experimental.pallas import tpu_sc as plsc

D = 768
B = 16384
SCALE = 1.0 / math.sqrt(768.0)

_NC = 2
_NS = 16
NW = _NC * _NS
B_PER_W = B // NW
SLOT = 64
CHUNK_SIZES = [32, 64, 64, 64, 64, 64, 64, 64, 32]
CHUNK_OFFS = [sum(CHUNK_SIZES[:i]) for i in range(len(CHUNK_SIZES))]
NCHUNK = len(CHUNK_SIZES)
D16 = D // 16

_mesh = plsc.VectorSubcoreMesh(core_axis_name="c", subcore_axis_name="s")


@functools.partial(
    pl.kernel,
    mesh=_mesh,
    out_type=jax.ShapeDtypeStruct((B, D), jnp.float32),
    scratch_types=[
        pltpu.VMEM((B_PER_W,), jnp.int32),
        pltpu.VMEM((SLOT, D), jnp.float32),
        pltpu.VMEM((SLOT, D), jnp.float32),
        pltpu.SemaphoreType.DMA,
        pltpu.SemaphoreType.DMA,
        pltpu.SemaphoreType.DMA,
        pltpu.SemaphoreType.DMA,
    ],
)
def _emb_kernel(x_hbm, table_hbm, out_hbm, idx_v, buf0, buf1,
                gs0, gs1, ss0, ss1):
    wid = lax.axis_index("s") * _NC + lax.axis_index("c")
    base = wid * B_PER_W
    pltpu.sync_copy(x_hbm.at[pl.ds(base, B_PER_W)], idx_v)

    bufs = (buf0, buf1)
    gsems = (gs0, gs1)
    ssems = (ss0, ss1)

    def start_gather(i):
        b = i % 2
        n = CHUNK_SIZES[i]
        return pltpu.async_copy(
            table_hbm.at[idx_v.at[pl.ds(CHUNK_OFFS[i], n)]],
            bufs[b].at[pl.ds(0, n)], gsems[b])

    def start_scatter(i, lo, n, sem):
        b = i % 2
        return pltpu.async_copy(
            bufs[b].at[pl.ds(lo, n)],
            out_hbm.at[pl.ds(base + CHUNK_OFFS[i] + lo, n)], sem)

    def scale(buf, lo, hi):
        def rows(t, carry):
            r = lo + t * 2
            for dr in range(2):
                for k in range(D16):
                    sl = (r + dr, pl.ds(k * 16, 16))
                    buf[sl] = buf[sl] * SCALE
            return carry
        lax.fori_loop(0, (hi - lo) // 2, rows, 0)

    g = [None] * NCHUNK
    s = [None] * NCHUNK
    g[0] = start_gather(0)
    for i in range(NCHUNK):
        b = i % 2
        if i + 1 < NCHUNK:
            if i >= 1:
                s[i - 1].wait()
            g[i + 1] = start_gather(i + 1)
        g[i].wait()
        if i == NCHUNK - 1:
            n = CHUNK_SIZES[i]
            half = n // 2
            scale(bufs[b], 0, half)
            sA = start_scatter(i, 0, half, ssems[b])
            scale(bufs[b], half, n)
            sB = start_scatter(i, half, n - half, gsems[b])
            s[NCHUNK - 2].wait()
            sA.wait()
            sB.wait()
        else:
            scale(bufs[b], 0, CHUNK_SIZES[i])
            s[i] = start_scatter(i, 0, CHUNK_SIZES[i], ssems[b])


def kernel(x, table):
    x_flat = x.reshape(-1).astype(jnp.int32)
    out = _emb_kernel(x_flat, table)
    return out.reshape(x.shape + (D,))


# speedup vs baseline: 1.1314x; 1.1314x over previous
import functools
import math

import jax
import jax.numpy as jnp
from jax import lax
from jax.experimental import pallas as pl
from jax.experimental.pallas import tpu as pltpu
from jax.experimental.pallas import tpu_sc as plsc

D = 768
B = 16384
SCALE = 1.0 / math.sqrt(768.0)

_NC = 2
_NS = 16
NW = _NC * _NS
B_PER_W = B // NW
SLOT = 64
CHUNK_SIZES = [32, 64, 64, 64, 64, 64, 64, 64, 32]
CHUNK_OFFS = [sum(CHUNK_SIZES[:i]) for i in range(len(CHUNK_SIZES))]
NCHUNK = len(CHUNK_SIZES)
D16 = D // 16

_mesh = plsc.VectorSubcoreMesh(core_axis_name="c", subcore_axis_name="s")


@functools.partial(
    pl.kernel,
    mesh=_mesh,
    out_type=jax.ShapeDtypeStruct((B, D), jnp.float32),
    scratch_types=[
        pltpu.VMEM((B_PER_W,), jnp.int32),
        pltpu.VMEM((SLOT, D), jnp.float32),
        pltpu.VMEM((SLOT, D), jnp.float32),
        pltpu.SemaphoreType.DMA,
        pltpu.SemaphoreType.DMA,
        pltpu.SemaphoreType.DMA,
        pltpu.SemaphoreType.DMA,
    ],
)
def _emb_kernel(x_hbm, table_hbm, out_hbm, idx_v, buf0, buf1,
                gs0, gs1, ss0, ss1):
    wid = lax.axis_index("s") * _NC + lax.axis_index("c")
    base = wid * B_PER_W
    pltpu.sync_copy(x_hbm.at[pl.ds(base, B_PER_W)], idx_v)

    bufs = (buf0, buf1)
    gsems = (gs0, gs1)
    ssems = (ss0, ss1)

    def start_gather(i):
        b = i % 2
        n = CHUNK_SIZES[i]
        return pltpu.async_copy(
            table_hbm.at[idx_v.at[pl.ds(CHUNK_OFFS[i], n)]],
            bufs[b].at[pl.ds(0, n)], gsems[b])

    def start_scatter(i, lo, n, sem):
        b = i % 2
        return pltpu.async_copy(
            bufs[b].at[pl.ds(lo, n)],
            out_hbm.at[pl.ds(base + CHUNK_OFFS[i] + lo, n)], sem)

    def scale(buf, lo, hi):
        def row(r, carry):
            for k in range(D16):
                sl = (r, pl.ds(k * 16, 16))
                buf[sl] = buf[sl] * SCALE
            return carry
        lax.fori_loop(lo, hi, row, 0)

    g = [None] * NCHUNK
    s = [None] * NCHUNK
    g[0] = start_gather(0)
    for i in range(NCHUNK):
        b = i % 2
        if i + 1 < NCHUNK:
            if i >= 1:
                s[i - 1].wait()
            g[i + 1] = start_gather(i + 1)
        g[i].wait()
        if i == NCHUNK - 1:
            n = CHUNK_SIZES[i]
            half = n // 2
            scale(bufs[b], 0, half)
            sA = start_scatter(i, 0, half, ssems[b])
            scale(bufs[b], half, n)
            sB = start_scatter(i, half, n - half, gsems[b])
            s[NCHUNK - 2].wait()
            sA.wait()
            sB.wait()
        else:
            scale(bufs[b], 0, CHUNK_SIZES[i])
            s[i] = start_scatter(i, 0, CHUNK_SIZES[i], ssems[b])


def kernel(x, table):
    x_flat = x.reshape(-1).astype(jnp.int32)
    out = _emb_kernel(x_flat, table)
    return out.reshape(x.shape + (D,))
